# baseline (speedup 1.0000x reference)
import jax
import jax.numpy as jnp
from jax import lax
from jax.experimental import pallas as pl
from jax.experimental.pallas import tpu as pltpu
from jax.experimental.pallas import tpu_sc as plsc

B, S, V, D = 4, 2048, 100000, 1024
NT = B * S
NC, NS = 2, 16
NW = NC * NS
TOK_PER_W = NT // NW
CH = 32
NCHUNK = TOK_PER_W // CH

MIN_F32 = float(jnp.finfo(jnp.float32).min)
MASK_BLK = 256


NBUF = 3


def _gather_body(idx_hbm, table_hbm, out_hbm, idx_v, *rest):
    wid = lax.axis_index("s") * NC + lax.axis_index("c")
    base = wid * TOK_PER_W
    pltpu.sync_copy(idx_hbm.at[wid], idx_v)
    bufs = rest[:NBUF]
    gsems = rest[NBUF:2 * NBUF]
    ssems = rest[2 * NBUF:3 * NBUF]
    g = [None] * NBUF
    s = [None] * NBUF
    for c in range(NCHUNK):
        b = c % NBUF
        if s[b] is not None:
            s[b].wait()
            s[b] = None
        g[b] = pltpu.async_copy(table_hbm.at[idx_v.at[c]], bufs[b], gsems[b])
        if c >= 1:
            pb = (c - 1) % NBUF
            g[pb].wait()
            s[pb] = pltpu.async_copy(
                bufs[pb], out_hbm.at[pl.ds(base + (c - 1) * CH, CH)], ssems[pb])
    lb = (NCHUNK - 1) % NBUF
    g[lb].wait()
    s[lb] = pltpu.async_copy(
        bufs[lb], out_hbm.at[pl.ds(base + (NCHUNK - 1) * CH, CH)], ssems[lb])
    for b in range(NBUF):
        if s[b] is not None:
            s[b].wait()


def _sc_gather(idx, table):
    mesh = plsc.VectorSubcoreMesh(core_axis_name="c", subcore_axis_name="s")
    k = pl.kernel(
        _gather_body,
        mesh=mesh,
        out_type=jax.ShapeDtypeStruct((NT, D), jnp.float32),
        cost_estimate=pl.CostEstimate(
            flops=0, transcendentals=0,
            bytes_accessed=2 * NT * D * 4 + NT * 4),
        scratch_types=(
            [pltpu.VMEM((NCHUNK, CH), jnp.int32)]
            + [pltpu.VMEM((CH, D), jnp.float32) for _ in range(NBUF)]
            + [pltpu.SemaphoreType.DMA for _ in range(2 * NBUF)]
        ),
    )
    return k(idx, table)


def _mask_body(am_ref, o_ref):
    r0 = pl.program_id(1) * MASK_BLK
    rows = lax.broadcasted_iota(jnp.int32, (MASK_BLK, S), 0) + r0
    cols = lax.broadcasted_iota(jnp.int32, (MASK_BLK, S), 1)
    inverted = jnp.where(am_ref[0, 0], 0.0, MIN_F32)
    o_ref[0, 0] = jnp.where(cols > rows, MIN_F32, inverted[None, :])


def _make_mask(attention_mask):
    am = attention_mask.reshape(B, 1, S)
    return pl.pallas_call(
        _mask_body,
        out_shape=jax.ShapeDtypeStruct((B, 1, S, S), jnp.float32),
        grid=(B, S // MASK_BLK),
        in_specs=[pl.BlockSpec((1, 1, S), lambda b, i: (b, 0, 0))],
        out_specs=pl.BlockSpec((1, 1, MASK_BLK, S), lambda b, i: (b, 0, i, 0)),
        cost_estimate=pl.CostEstimate(
            flops=2 * B * S * S, transcendentals=0,
            bytes_accessed=B * S * S * 4),
    )(am)


def kernel(input_ids, attention_mask, position_ids, labels, W):
    idx = input_ids.reshape(NW, NCHUNK, CH)
    hidden = _sc_gather(idx, W).reshape(B, S, D)
    mask = _make_mask(attention_mask)
    return (hidden, mask, position_ids, labels)


# speedup vs baseline: 1.0199x; 1.0199x over previous
import jax
import jax.numpy as jnp
from jax import lax
from jax.experimental import pallas as pl
from jax.experimental.pallas import tpu as pltpu
from jax.experimental.pallas import tpu_sc as plsc

B, S, V, D = 4, 2048, 100000, 1024
NT = B * S
NC, NS = 2, 16
NW = NC * NS
TOK_PER_W = NT // NW
CH = 32
NCHUNK = TOK_PER_W // CH

MIN_F32 = float(jnp.finfo(jnp.float32).min)
MASK_BLK = 512


NBUF = 3


def _gather_body(idx_hbm, table_hbm, out_hbm, idx_v, *rest):
    wid = lax.axis_index("s") * NC + lax.axis_index("c")
    base = wid * TOK_PER_W
    bufs = rest[:NBUF]
    gsems = rest[NBUF:2 * NBUF]
    ssems = rest[2 * NBUF:3 * NBUF]
    pltpu.sync_copy(idx_hbm.at[wid].at[pl.ds(0, 1)], idx_v.at[pl.ds(0, 1)])
    g = [None] * NBUF
    s = [None] * NBUF
    for c in range(NCHUNK):
        b = c % NBUF
        if s[b] is not None:
            s[b].wait()
            s[b] = None
        g[b] = pltpu.async_copy(table_hbm.at[idx_v.at[c]], bufs[b], gsems[b])
        if c == 0:
            pltpu.sync_copy(idx_hbm.at[wid].at[pl.ds(1, NCHUNK - 1)],
                            idx_v.at[pl.ds(1, NCHUNK - 1)])
        if c >= 1:
            pb = (c - 1) % NBUF
            g[pb].wait()
            s[pb] = pltpu.async_copy(
                bufs[pb], out_hbm.at[pl.ds(base + (c - 1) * CH, CH)], ssems[pb])
    lb = (NCHUNK - 1) % NBUF
    g[lb].wait()
    s[lb] = pltpu.async_copy(
        bufs[lb], out_hbm.at[pl.ds(base + (NCHUNK - 1) * CH, CH)], ssems[lb])
    for b in range(NBUF):
        if s[b] is not None:
            s[b].wait()


def _sc_gather(idx, table):
    mesh = plsc.VectorSubcoreMesh(core_axis_name="c", subcore_axis_name="s")
    k = pl.kernel(
        _gather_body,
        mesh=mesh,
        out_type=jax.ShapeDtypeStruct((NT, D), jnp.float32),
        cost_estimate=pl.CostEstimate(
            flops=0, transcendentals=0,
            bytes_accessed=2 * NT * D * 4 + NT * 4),
        scratch_types=(
            [pltpu.VMEM((NCHUNK, CH), jnp.int32)]
            + [pltpu.VMEM((CH, D), jnp.float32) for _ in range(NBUF)]
            + [pltpu.SemaphoreType.DMA for _ in range(2 * NBUF)]
        ),
    )
    return k(idx, table)


def _mask_body(am_ref, o_ref):
    r0 = pl.program_id(1) * MASK_BLK
    rows = lax.broadcasted_iota(jnp.int32, (MASK_BLK, S), 0) + r0
    cols = lax.broadcasted_iota(jnp.int32, (MASK_BLK, S), 1)
    inverted = jnp.where(am_ref[0, 0], 0.0, MIN_F32)
    o_ref[0, 0] = jnp.where(cols > rows, MIN_F32, inverted[None, :])


def _make_mask(attention_mask):
    am = attention_mask.reshape(B, 1, S)
    return pl.pallas_call(
        _mask_body,
        out_shape=jax.ShapeDtypeStruct((B, 1, S, S), jnp.float32),
        grid=(B, S // MASK_BLK),
        in_specs=[pl.BlockSpec((1, 1, S), lambda b, i: (b, 0, 0))],
        out_specs=pl.BlockSpec((1, 1, MASK_BLK, S), lambda b, i: (b, 0, i, 0)),
        cost_estimate=pl.CostEstimate(
            flops=2 * B * S * S, transcendentals=0,
            bytes_accessed=B * S * S * 4),
    )(am)


def kernel(input_ids, attention_mask, position_ids, labels, W):
    idx = input_ids.reshape(NW, NCHUNK, CH)
    hidden = _sc_gather(idx, W).reshape(B, S, D)
    mask = _make_mask(attention_mask)
    return (hidden, mask, position_ids, labels)
